# baseline (speedup 1.0000x reference)
import jax
import jax.numpy as jnp
from jax import lax
from jax.experimental import pallas as pl
from jax.experimental.pallas import tpu as pltpu
from jax.experimental.pallas import tpu_sc as plsc

DIM = 1024
SEQ_LEN = 4096
SCALE = DIM ** (-0.5)

_INFO = plsc.get_sparse_core_info()
NC = _INFO.num_cores
NS = _INFO.num_subcores
LANES = _INFO.num_lanes
NW = NC * NS

ROWS_PER_W = SEQ_LEN // NW
CHUNK_ROWS = 8
NCHUNK = ROWS_PER_W // CHUNK_ROWS
SLICES_PER_ROW = DIM // LANES

NBUF = 15
PF = 12


def _scale_chunk(buf):
    def body(i, c):
        r = i >> 6
        col = (i & (SLICES_PER_ROW - 1)) * LANES
        sl = pl.ds(col, LANES)
        buf[r, sl] = buf[r, sl] * SCALE
        return c

    plsc.parallel_loop(
        0, CHUNK_ROWS * SLICES_PER_ROW, 1, unroll=8, carry=jnp.int32(0))(body)


def _body(table_hbm, out_hbm, *scratch):
    bufs = scratch[:NBUF]
    sin = scratch[NBUF:2 * NBUF]
    sout = scratch[2 * NBUF:3 * NBUF]
    wid = lax.axis_index("s") * NC + lax.axis_index("c")
    base_row = wid * ROWS_PER_W

    def start_in(g):
        row = base_row + g * CHUNK_ROWS
        return pltpu.async_copy(
            table_hbm.at[pl.ds(row, CHUNK_ROWS), :], bufs[g % NBUF],
            sin[g % NBUF])

    def start_out(g):
        row = base_row + g * CHUNK_ROWS
        return pltpu.async_copy(
            bufs[g % NBUF], out_hbm.at[pl.ds(row, CHUNK_ROWS), :],
            sout[g % NBUF])

    in_h, out_h, waited = {}, {}, set()
    for g in range(min(PF, NCHUNK)):
        in_h[g] = start_in(g)
    for g in range(NCHUNK):
        n = g + PF
        if n < NCHUNK:
            prev = n - NBUF
            if prev >= 0:
                out_h[prev].wait()
                waited.add(prev)
            in_h[n] = start_in(n)
        in_h[g].wait()
        _scale_chunk(bufs[g % NBUF])
        out_h[g] = start_out(g)
    for g in range(NCHUNK):
        if g not in waited:
            out_h[g].wait()


@jax.jit
def _sc_scale(table):
    mesh = plsc.VectorSubcoreMesh(core_axis_name="c", subcore_axis_name="s")
    fn = pl.kernel(
        _body,
        out_type=jax.ShapeDtypeStruct((SEQ_LEN, DIM), jnp.float32),
        mesh=mesh,
        scratch_types=(
            [pltpu.VMEM((CHUNK_ROWS, DIM), jnp.float32) for _ in range(NBUF)]
            + [pltpu.SemaphoreType.DMA for _ in range(2 * NBUF)]
        ),
    )
    return fn(table)


def kernel(x, table):
    del x
    return _sc_scale(table)


# speedup vs baseline: 1.0486x; 1.0486x over previous
import jax
import jax.numpy as jnp
from jax import lax
from jax.experimental import pallas as pl
from jax.experimental.pallas import tpu as pltpu
from jax.experimental.pallas import tpu_sc as plsc

DIM = 1024
SEQ_LEN = 4096
SCALE = DIM ** (-0.5)

_INFO = plsc.get_sparse_core_info()
NC = _INFO.num_cores
NS = _INFO.num_subcores
LANES = _INFO.num_lanes
NW = NC * NS

ROWS_PER_W = SEQ_LEN // NW
CHUNK_ROWS = 16
NCHUNK = ROWS_PER_W // CHUNK_ROWS
SLICES_PER_ROW = DIM // LANES

NBUF = 7
PF = 6


def _scale_chunk(buf):
    def body(i, c):
        r = i >> 6
        col = (i & (SLICES_PER_ROW - 1)) * LANES
        sl = pl.ds(col, LANES)
        buf[r, sl] = buf[r, sl] * SCALE
        return c

    plsc.parallel_loop(
        0, CHUNK_ROWS * SLICES_PER_ROW, 1, unroll=8, carry=jnp.int32(0))(body)


def _body(table_hbm, out_hbm, *scratch):
    bufs = scratch[:NBUF]
    sin = scratch[NBUF:2 * NBUF]
    sout = scratch[2 * NBUF:3 * NBUF]
    wid = lax.axis_index("s") * NC + lax.axis_index("c")
    base_row = wid * ROWS_PER_W

    def start_in(g):
        row = base_row + g * CHUNK_ROWS
        return pltpu.async_copy(
            table_hbm.at[pl.ds(row, CHUNK_ROWS), :], bufs[g % NBUF],
            sin[g % NBUF])

    def start_out(g):
        row = base_row + g * CHUNK_ROWS
        return pltpu.async_copy(
            bufs[g % NBUF], out_hbm.at[pl.ds(row, CHUNK_ROWS), :],
            sout[g % NBUF])

    in_h, out_h, waited = {}, {}, set()
    for g in range(min(PF, NCHUNK)):
        in_h[g] = start_in(g)
    for g in range(NCHUNK):
        n = g + PF
        if n < NCHUNK:
            prev = n - NBUF
            if prev >= 0:
                out_h[prev].wait()
                waited.add(prev)
            in_h[n] = start_in(n)
        in_h[g].wait()
        _scale_chunk(bufs[g % NBUF])
        out_h[g] = start_out(g)
    for g in range(NCHUNK):
        if g not in waited:
            out_h[g].wait()


@jax.jit
def _sc_scale(table):
    mesh = plsc.VectorSubcoreMesh(core_axis_name="c", subcore_axis_name="s")
    fn = pl.kernel(
        _body,
        out_type=jax.ShapeDtypeStruct((SEQ_LEN, DIM), jnp.float32),
        mesh=mesh,
        scratch_types=(
            [pltpu.VMEM((CHUNK_ROWS, DIM), jnp.float32) for _ in range(NBUF)]
            + [pltpu.SemaphoreType.DMA for _ in range(2 * NBUF)]
        ),
    )
    return fn(table)


def kernel(x, table):
    del x
    return _sc_scale(table)
